# baseline (speedup 1.0000x reference)
import jax
import jax.numpy as jnp
from jax.experimental import pallas as pl
from jax.experimental.pallas import tpu as pltpu


def _update_body(pos_ref, kc_ref, vc_ref, kv_ref, vv_ref, ko_ref, vo_ref):
    ko_ref[...] = kc_ref[...]
    vo_ref[...] = vc_ref[...]
    q = kv_ref.shape[1]
    for i in range(q):
        p = pos_ref[i]
        ko_ref[0, p, :] = kv_ref[0, i, :]
        vo_ref[0, p, :] = vv_ref[0, i, :]


def kernel(input_pos, k_val, v_val, k_cache, v_cache):
    B, H, S, D = k_cache.shape
    Q = k_val.shape[2]
    BH = B * H
    kc = k_cache.reshape(BH, S, D)
    vc = v_cache.reshape(BH, S, D)
    kv = k_val.reshape(BH, Q, D)
    vv = v_val.reshape(BH, Q, D)
    ko, vo = pl.pallas_call(
        _update_body,
        grid=(BH,),
        in_specs=[
            pl.BlockSpec(memory_space=pltpu.SMEM),
            pl.BlockSpec((1, S, D), lambda i: (i, 0, 0)),
            pl.BlockSpec((1, S, D), lambda i: (i, 0, 0)),
            pl.BlockSpec((1, Q, D), lambda i: (i, 0, 0)),
            pl.BlockSpec((1, Q, D), lambda i: (i, 0, 0)),
        ],
        out_specs=[
            pl.BlockSpec((1, S, D), lambda i: (i, 0, 0)),
            pl.BlockSpec((1, S, D), lambda i: (i, 0, 0)),
        ],
        out_shape=[jax.ShapeDtypeStruct((BH, S, D), jnp.float32)] * 2,
        compiler_params=pltpu.CompilerParams(
            dimension_semantics=("arbitrary",)
        ),
    )(input_pos.astype(jnp.int32), kc, vc, kv, vv)
    return ko.reshape(B, H, S, D), vo.reshape(B, H, S, D)
